# baseline (speedup 1.0000x reference)
import jax
import jax.numpy as jnp
from jax import lax
from jax.experimental import pallas as pl
from jax.experimental.pallas import tpu as pltpu

POS_RADIUS2 = 4.0
NEG_RADIUS2 = 6.25
POS_MARGIN = 0.1
NEG_MARGIN = 1.4

_BB = 4


def _body(tr, refp, srcp, refm, srcm, refs, srcs, match, reff, srcf, out):
    g = pl.program_id(0)

    R = [[tr[l, k] for k in range(3)] for l in range(3)]
    t = [tr[l, 3] for l in range(3)]

    rp = [refp[k] for k in range(3)]
    sp_in = [srcp[k] for k in range(3)]
    sp = [sp_in[0] * R[l][0] + sp_in[1] * R[l][1] + sp_in[2] * R[l][2] + t[l]
          for l in range(3)]
    sq_r = rp[0] * rp[0] + rp[1] * rp[1] + rp[2] * rp[2]
    sq_s = sp[0] * sp[0] + sp[1] * sp[1] + sp[2] * sp[2]
    ab = (rp[0][..., :, None] * sp[0][..., None, :]
          + rp[1][..., :, None] * sp[1][..., None, :]
          + rp[2][..., :, None] * sp[2][..., None, :])
    d = sq_r[..., :, None] + sq_s[..., None, :] - 2.0 * ab

    refm_v = refm[...]
    srcm_v = srcm[...]
    gtm = refm_v[..., :, None] * srcm_v[..., None, :]
    gt = jnp.where(d < POS_RADIUS2, gtm, 0.0)
    neg = jnp.where(d > NEG_RADIUS2, gtm, 0.0)

    row = jnp.sum(gt, axis=3)
    col = jnp.sum(gt, axis=2)
    slack_r = jnp.where(row == 0.0, refm_v, 0.0)
    slack_c = jnp.where(col == 0.0, srcm_v, 0.0)

    s1 = jnp.sum(jnp.log(jnp.where(gt > 0.0, match[...], 1.0)))
    c1 = jnp.sum(gt)
    s2 = jnp.sum(jnp.log(jnp.where(slack_r > 0.0, 1.0 - refs[...], 1.0)))
    c2 = jnp.sum(slack_r)
    s3 = jnp.sum(jnp.log(jnp.where(slack_c > 0.0, 1.0 - srcs[...], 1.0)))
    c3 = jnp.sum(slack_c)
    cneg = jnp.sum(neg)

    BBk, _, K, C3 = reff.shape
    C = C3 // 3
    ri = lax.broadcasted_iota(jnp.int32, (C3, C), 0)
    ci = lax.broadcasted_iota(jnp.int32, (C3, C), 1)
    sel_k = [(ri == 3 * ci + k).astype(jnp.float32) for k in range(3)]
    rot_l = [sel_k[0] * R[l][0] + sel_k[1] * R[l][1] + sel_k[2] * R[l][2]
             for l in range(3)]
    xr = reff[...].reshape(BBk * 2 * K, C3)
    xs = srcf[...].reshape(BBk * 2 * K, C3)
    dn2 = (((1,), (0,)), ((), ()))

    def deinterleave(x, sel):
        y = lax.dot_general(x, sel, dn2, preferred_element_type=jnp.float32)
        y = y.reshape(BBk, 2, K, C)
        return jnp.concatenate([y[:, 0], y[:, 1]], axis=-1)

    rf = [deinterleave(xr, sel_k[k]) for k in range(3)]
    sf = [deinterleave(xs, rot_l[l]) for l in range(3)]
    d0 = rf[0][:, :, None, :] - sf[0][:, None, :, :]
    d1 = rf[1][:, :, None, :] - sf[1][:, None, :, :]
    d2 = rf[2][:, :, None, :] - sf[2][:, None, :, :]
    sqd = d0 * d0 + d1 * d1 + d2 * d2
    norms = sqd * lax.rsqrt(jnp.maximum(sqd, 1e-12))

    BB, K, _, C2 = sqd.shape
    C = C2 // 2
    hp = jnp.maximum(norms - POS_MARGIN, 0.0).reshape(BB, K * K, C2)
    hn = jnp.maximum(NEG_MARGIN - norms, 0.0).reshape(BB, K * K, C2)
    gtf = gt.reshape(BB, 2, K * K).astype(jnp.bfloat16)
    negf = neg.reshape(BB, 2, K * K).astype(jnp.bfloat16)
    dn = (((2,), (1,)), ((0,), (0,)))
    P = lax.dot_general(gtf, hp.astype(jnp.bfloat16), dn,
                        preferred_element_type=jnp.float32)
    N = lax.dot_general(negf, hn.astype(jnp.bfloat16), dn,
                        preferred_element_type=jnp.float32)
    lanei = lax.broadcasted_iota(jnp.int32, (BB, 2, C2), 2)
    rowi = lax.broadcasted_iota(jnp.int32, (BB, 2, C2), 1)
    sel = (lanei // C) == rowi
    pos_s = jnp.sum(jnp.where(sel, P, 0.0))
    neg_s = jnp.sum(jnp.where(sel, N, 0.0))

    @pl.when(g == 0)
    def _():
        out[...] = jnp.zeros_like(out)

    out[0:1, :] = out[0:1, :] + s1
    out[1:2, :] = out[1:2, :] + c1
    out[2:3, :] = out[2:3, :] + s2
    out[3:4, :] = out[3:4, :] + c2
    out[4:5, :] = out[4:5, :] + s3
    out[5:6, :] = out[5:6, :] + c3
    out[6:7, :] = out[6:7, :] + pos_s
    out[7:8, :] = out[7:8, :] + neg_s
    out[8:9, :] = out[8:9, :] + cneg


def kernel(ref_node_corr_knn_points, src_node_corr_knn_points,
           ref_node_corr_knn_masks, src_node_corr_knn_masks,
           ref_node_corr_knn_scores, src_node_corr_knn_scores,
           matching_scores, transform,
           re_ref_node_corr_knn_feats, re_src_node_corr_knn_feats):
    B, K, C, _ = re_ref_node_corr_knn_feats.shape
    B2 = B // 2
    G = B2 // _BB

    def fold_feats(f):
        return f.reshape(B2, 2, K, 3 * C)

    refp = ref_node_corr_knn_points.transpose(2, 0, 1).reshape(3, B2, 2, K)
    srcp = src_node_corr_knn_points.transpose(2, 0, 1).reshape(3, B2, 2, K)
    refm = ref_node_corr_knn_masks.astype(jnp.float32).reshape(B2, 2, K)
    srcm = src_node_corr_knn_masks.astype(jnp.float32).reshape(B2, 2, K)
    refs = ref_node_corr_knn_scores.reshape(B2, 2, K)
    srcs = src_node_corr_knn_scores.reshape(B2, 2, K)
    match = matching_scores.reshape(B2, 2, K, K)
    reff = fold_feats(re_ref_node_corr_knn_feats)
    srcf = fold_feats(re_src_node_corr_knn_feats)

    out = pl.pallas_call(
        _body,
        grid=(G,),
        in_specs=[
            pl.BlockSpec(memory_space=pltpu.SMEM),
            pl.BlockSpec((3, _BB, 2, K), lambda g: (0, g, 0, 0)),
            pl.BlockSpec((3, _BB, 2, K), lambda g: (0, g, 0, 0)),
            pl.BlockSpec((_BB, 2, K), lambda g: (g, 0, 0)),
            pl.BlockSpec((_BB, 2, K), lambda g: (g, 0, 0)),
            pl.BlockSpec((_BB, 2, K), lambda g: (g, 0, 0)),
            pl.BlockSpec((_BB, 2, K), lambda g: (g, 0, 0)),
            pl.BlockSpec((_BB, 2, K, K), lambda g: (g, 0, 0, 0)),
            pl.BlockSpec((_BB, 2, K, 3 * C), lambda g: (g, 0, 0, 0)),
            pl.BlockSpec((_BB, 2, K, 3 * C), lambda g: (g, 0, 0, 0)),
        ],
        out_specs=pl.BlockSpec((16, 128), lambda g: (0, 0)),
        out_shape=jax.ShapeDtypeStruct((16, 128), jnp.float32),
        compiler_params=pltpu.CompilerParams(
            dimension_semantics=("arbitrary",)),
    )(transform, refp, srcp, refm, srcm, refs, srcs, match, reff, srcf)

    v = out[:, 0]
    s1, c1, s2, c2, s3, c3, pos_s, neg_s, cneg = (
        v[0], v[1], v[2], v[3], v[4], v[5], v[6], v[7], v[8])
    fine_ri_loss = -(s1 / jnp.maximum(c1, 1.0)
                     + 0.5 * s2 / jnp.maximum(c2, 1.0)
                     + 0.5 * s3 / jnp.maximum(c3, 1.0))
    Cf = jnp.float32(C)
    pos_loss = pos_s / jnp.maximum(c1 * Cf, 1.0)
    neg_loss = neg_s / jnp.maximum(cneg * Cf, 1.0)
    fine_re_loss = jnp.where(c1 == 0.0, 0.0, pos_loss + neg_loss)
    return (fine_ri_loss, fine_re_loss)


# speedup vs baseline: 1.2307x; 1.2307x over previous
import jax
import jax.numpy as jnp
from jax import lax
from jax.experimental import pallas as pl
from jax.experimental.pallas import tpu as pltpu

POS_RADIUS2 = 4.0
NEG_RADIUS2 = 6.25
POS_MARGIN = 0.1
NEG_MARGIN = 1.4

_BB = 4


def _body(tr, refp, srcp, refm, srcm, refs, srcs, match, reff, srcf, out):
    g = pl.program_id(0)

    R = [[tr[l, k] for k in range(3)] for l in range(3)]
    t = [tr[l, 3] for l in range(3)]

    rp = [refp[k] for k in range(3)]
    sp_in = [srcp[k] for k in range(3)]
    sp = [sp_in[0] * R[l][0] + sp_in[1] * R[l][1] + sp_in[2] * R[l][2] + t[l]
          for l in range(3)]
    sq_r = rp[0] * rp[0] + rp[1] * rp[1] + rp[2] * rp[2]
    sq_s = sp[0] * sp[0] + sp[1] * sp[1] + sp[2] * sp[2]
    ab = (rp[0][..., :, None] * sp[0][..., None, :]
          + rp[1][..., :, None] * sp[1][..., None, :]
          + rp[2][..., :, None] * sp[2][..., None, :])
    d = sq_r[..., :, None] + sq_s[..., None, :] - 2.0 * ab

    refm_v = refm[...]
    srcm_v = srcm[...]
    gtm = refm_v[..., :, None] * srcm_v[..., None, :]
    gt = jnp.where(d < POS_RADIUS2, gtm, 0.0)
    neg = jnp.where(d > NEG_RADIUS2, gtm, 0.0)

    row = jnp.sum(gt, axis=3)
    col = jnp.sum(gt, axis=2)
    slack_r = jnp.where(row == 0.0, refm_v, 0.0)
    slack_c = jnp.where(col == 0.0, srcm_v, 0.0)

    s1 = jnp.sum(jnp.log(jnp.where(gt > 0.0, match[...], 1.0)))
    c1 = jnp.sum(gt)
    s2 = jnp.sum(jnp.log(jnp.where(slack_r > 0.0, 1.0 - refs[...], 1.0)))
    c2 = jnp.sum(slack_r)
    s3 = jnp.sum(jnp.log(jnp.where(slack_c > 0.0, 1.0 - srcs[...], 1.0)))
    c3 = jnp.sum(slack_c)
    cneg = jnp.sum(neg)

    rf = [reff[k] for k in range(3)]
    sf_in = [srcf[k] for k in range(3)]
    sf = [sf_in[0] * R[l][0] + sf_in[1] * R[l][1] + sf_in[2] * R[l][2]
          for l in range(3)]
    d0 = rf[0][:, :, None, :] - sf[0][:, None, :, :]
    d1 = rf[1][:, :, None, :] - sf[1][:, None, :, :]
    d2 = rf[2][:, :, None, :] - sf[2][:, None, :, :]
    sqd = d0 * d0 + d1 * d1 + d2 * d2
    norms = sqd * lax.rsqrt(jnp.maximum(sqd, 1e-12))

    BB, K, _, C2 = sqd.shape
    C = C2 // 2
    hp = jnp.maximum(norms - POS_MARGIN, 0.0).reshape(BB, K * K, C2)
    hn = jnp.maximum(NEG_MARGIN - norms, 0.0).reshape(BB, K * K, C2)
    gtf = gt.reshape(BB, 2, K * K).astype(jnp.bfloat16)
    negf = neg.reshape(BB, 2, K * K).astype(jnp.bfloat16)
    dn = (((2,), (1,)), ((0,), (0,)))
    P = lax.dot_general(gtf, hp.astype(jnp.bfloat16), dn,
                        preferred_element_type=jnp.float32)
    N = lax.dot_general(negf, hn.astype(jnp.bfloat16), dn,
                        preferred_element_type=jnp.float32)
    lanei = lax.broadcasted_iota(jnp.int32, (BB, 2, C2), 2)
    rowi = lax.broadcasted_iota(jnp.int32, (BB, 2, C2), 1)
    sel = (lanei // C) == rowi
    pos_s = jnp.sum(jnp.where(sel, P, 0.0))
    neg_s = jnp.sum(jnp.where(sel, N, 0.0))

    @pl.when(g == 0)
    def _():
        out[...] = jnp.zeros_like(out)

    out[0:1, :] = out[0:1, :] + s1
    out[1:2, :] = out[1:2, :] + c1
    out[2:3, :] = out[2:3, :] + s2
    out[3:4, :] = out[3:4, :] + c2
    out[4:5, :] = out[4:5, :] + s3
    out[5:6, :] = out[5:6, :] + c3
    out[6:7, :] = out[6:7, :] + pos_s
    out[7:8, :] = out[7:8, :] + neg_s
    out[8:9, :] = out[8:9, :] + cneg


def kernel(ref_node_corr_knn_points, src_node_corr_knn_points,
           ref_node_corr_knn_masks, src_node_corr_knn_masks,
           ref_node_corr_knn_scores, src_node_corr_knn_scores,
           matching_scores, transform,
           re_ref_node_corr_knn_feats, re_src_node_corr_knn_feats):
    B, K, C, _ = re_ref_node_corr_knn_feats.shape
    B2 = B // 2
    G = B2 // _BB

    def fold_feats(f):
        return (f.reshape(B2, 2, K, C, 3)
                 .transpose(4, 0, 2, 1, 3)
                 .reshape(3, B2, K, 2 * C))

    refp = ref_node_corr_knn_points.transpose(2, 0, 1).reshape(3, B2, 2, K)
    srcp = src_node_corr_knn_points.transpose(2, 0, 1).reshape(3, B2, 2, K)
    refm = ref_node_corr_knn_masks.astype(jnp.float32).reshape(B2, 2, K)
    srcm = src_node_corr_knn_masks.astype(jnp.float32).reshape(B2, 2, K)
    refs = ref_node_corr_knn_scores.reshape(B2, 2, K)
    srcs = src_node_corr_knn_scores.reshape(B2, 2, K)
    match = matching_scores.reshape(B2, 2, K, K)
    reff = fold_feats(re_ref_node_corr_knn_feats)
    srcf = fold_feats(re_src_node_corr_knn_feats)

    out = pl.pallas_call(
        _body,
        grid=(G,),
        in_specs=[
            pl.BlockSpec(memory_space=pltpu.SMEM),
            pl.BlockSpec((3, _BB, 2, K), lambda g: (0, g, 0, 0)),
            pl.BlockSpec((3, _BB, 2, K), lambda g: (0, g, 0, 0)),
            pl.BlockSpec((_BB, 2, K), lambda g: (g, 0, 0)),
            pl.BlockSpec((_BB, 2, K), lambda g: (g, 0, 0)),
            pl.BlockSpec((_BB, 2, K), lambda g: (g, 0, 0)),
            pl.BlockSpec((_BB, 2, K), lambda g: (g, 0, 0)),
            pl.BlockSpec((_BB, 2, K, K), lambda g: (g, 0, 0, 0)),
            pl.BlockSpec((3, _BB, K, 2 * C), lambda g: (0, g, 0, 0)),
            pl.BlockSpec((3, _BB, K, 2 * C), lambda g: (0, g, 0, 0)),
        ],
        out_specs=pl.BlockSpec((16, 128), lambda g: (0, 0)),
        out_shape=jax.ShapeDtypeStruct((16, 128), jnp.float32),
        compiler_params=pltpu.CompilerParams(
            dimension_semantics=("arbitrary",)),
    )(transform, refp, srcp, refm, srcm, refs, srcs, match, reff, srcf)

    v = out[:, 0]
    s1, c1, s2, c2, s3, c3, pos_s, neg_s, cneg = (
        v[0], v[1], v[2], v[3], v[4], v[5], v[6], v[7], v[8])
    fine_ri_loss = -(s1 / jnp.maximum(c1, 1.0)
                     + 0.5 * s2 / jnp.maximum(c2, 1.0)
                     + 0.5 * s3 / jnp.maximum(c3, 1.0))
    Cf = jnp.float32(C)
    pos_loss = pos_s / jnp.maximum(c1 * Cf, 1.0)
    neg_loss = neg_s / jnp.maximum(cneg * Cf, 1.0)
    fine_re_loss = jnp.where(c1 == 0.0, 0.0, pos_loss + neg_loss)
    return (fine_ri_loss, fine_re_loss)


# speedup vs baseline: 1.5241x; 1.2383x over previous
import jax
import jax.numpy as jnp
from jax import lax
from jax.experimental import pallas as pl
from jax.experimental.pallas import tpu as pltpu

POS_RADIUS2 = 4.0
NEG_RADIUS2 = 6.25
POS_MARGIN = 0.1
NEG_MARGIN = 1.4

_BB = 4


def _body(tr, refp, srcp, refm, srcm, refs, srcs, match, reff, srcf, out):
    g = pl.program_id(0)

    R = [[tr[l, k] for k in range(3)] for l in range(3)]
    t = [tr[l, 3] for l in range(3)]

    rp = [refp[k] for k in range(3)]
    sp_in = [srcp[k] for k in range(3)]
    sp = [sp_in[0] * R[l][0] + sp_in[1] * R[l][1] + sp_in[2] * R[l][2] + t[l]
          for l in range(3)]
    sq_r = rp[0] * rp[0] + rp[1] * rp[1] + rp[2] * rp[2]
    sq_s = sp[0] * sp[0] + sp[1] * sp[1] + sp[2] * sp[2]
    ab = (rp[0][..., :, None] * sp[0][..., None, :]
          + rp[1][..., :, None] * sp[1][..., None, :]
          + rp[2][..., :, None] * sp[2][..., None, :])
    d = sq_r[..., :, None] + sq_s[..., None, :] - 2.0 * ab

    refm_v = refm[...]
    srcm_v = srcm[...]
    gtm = refm_v[..., :, None] * srcm_v[..., None, :]
    gt = jnp.where(d < POS_RADIUS2, gtm, 0.0)
    neg = jnp.where(d > NEG_RADIUS2, gtm, 0.0)

    row = jnp.sum(gt, axis=3)
    col = jnp.sum(gt, axis=2)
    slack_r = jnp.where(row == 0.0, refm_v, 0.0)
    slack_c = jnp.where(col == 0.0, srcm_v, 0.0)

    s1 = jnp.sum(jnp.log(jnp.where(gt > 0.0, match[...], 1.0)))
    c1 = jnp.sum(gt)
    s2 = jnp.sum(jnp.log(jnp.where(slack_r > 0.0, 1.0 - refs[...], 1.0)))
    c2 = jnp.sum(slack_r)
    s3 = jnp.sum(jnp.log(jnp.where(slack_c > 0.0, 1.0 - srcs[...], 1.0)))
    c3 = jnp.sum(slack_c)
    cneg = jnp.sum(neg)

    rf = [jnp.concatenate([reff[k, :, 0], reff[k, :, 1]], axis=-1)
          for k in range(3)]
    sf_in = [jnp.concatenate([srcf[k, :, 0], srcf[k, :, 1]], axis=-1)
             for k in range(3)]
    sf = [sf_in[0] * R[l][0] + sf_in[1] * R[l][1] + sf_in[2] * R[l][2]
          for l in range(3)]
    d0 = rf[0][:, :, None, :] - sf[0][:, None, :, :]
    d1 = rf[1][:, :, None, :] - sf[1][:, None, :, :]
    d2 = rf[2][:, :, None, :] - sf[2][:, None, :, :]
    sqd = d0 * d0 + d1 * d1 + d2 * d2
    norms = sqd * lax.rsqrt(jnp.maximum(sqd, 1e-12))

    BB, K, _, C2 = sqd.shape
    C = C2 // 2
    hp = jnp.maximum(norms - POS_MARGIN, 0.0).reshape(BB, K * K, C2)
    hn = jnp.maximum(NEG_MARGIN - norms, 0.0).reshape(BB, K * K, C2)
    gtf = gt.reshape(BB, 2, K * K).astype(jnp.bfloat16)
    negf = neg.reshape(BB, 2, K * K).astype(jnp.bfloat16)
    dn = (((2,), (1,)), ((0,), (0,)))
    P = lax.dot_general(gtf, hp.astype(jnp.bfloat16), dn,
                        preferred_element_type=jnp.float32)
    N = lax.dot_general(negf, hn.astype(jnp.bfloat16), dn,
                        preferred_element_type=jnp.float32)
    lanei = lax.broadcasted_iota(jnp.int32, (BB, 2, C2), 2)
    rowi = lax.broadcasted_iota(jnp.int32, (BB, 2, C2), 1)
    sel = (lanei // C) == rowi
    pos_s = jnp.sum(jnp.where(sel, P, 0.0))
    neg_s = jnp.sum(jnp.where(sel, N, 0.0))

    @pl.when(g == 0)
    def _():
        out[...] = jnp.zeros_like(out)

    out[0:1, :] = out[0:1, :] + s1
    out[1:2, :] = out[1:2, :] + c1
    out[2:3, :] = out[2:3, :] + s2
    out[3:4, :] = out[3:4, :] + c2
    out[4:5, :] = out[4:5, :] + s3
    out[5:6, :] = out[5:6, :] + c3
    out[6:7, :] = out[6:7, :] + pos_s
    out[7:8, :] = out[7:8, :] + neg_s
    out[8:9, :] = out[8:9, :] + cneg


def kernel(ref_node_corr_knn_points, src_node_corr_knn_points,
           ref_node_corr_knn_masks, src_node_corr_knn_masks,
           ref_node_corr_knn_scores, src_node_corr_knn_scores,
           matching_scores, transform,
           re_ref_node_corr_knn_feats, re_src_node_corr_knn_feats):
    B, K, C, _ = re_ref_node_corr_knn_feats.shape
    B2 = B // 2
    G = B2 // _BB

    def fold_feats(f):
        return f.transpose(3, 0, 1, 2).reshape(3, B2, 2, K, C)

    refp = ref_node_corr_knn_points.transpose(2, 0, 1).reshape(3, B2, 2, K)
    srcp = src_node_corr_knn_points.transpose(2, 0, 1).reshape(3, B2, 2, K)
    refm = ref_node_corr_knn_masks.astype(jnp.float32).reshape(B2, 2, K)
    srcm = src_node_corr_knn_masks.astype(jnp.float32).reshape(B2, 2, K)
    refs = ref_node_corr_knn_scores.reshape(B2, 2, K)
    srcs = src_node_corr_knn_scores.reshape(B2, 2, K)
    match = matching_scores.reshape(B2, 2, K, K)
    reff = fold_feats(re_ref_node_corr_knn_feats)
    srcf = fold_feats(re_src_node_corr_knn_feats)

    out = pl.pallas_call(
        _body,
        grid=(G,),
        in_specs=[
            pl.BlockSpec(memory_space=pltpu.SMEM),
            pl.BlockSpec((3, _BB, 2, K), lambda g: (0, g, 0, 0)),
            pl.BlockSpec((3, _BB, 2, K), lambda g: (0, g, 0, 0)),
            pl.BlockSpec((_BB, 2, K), lambda g: (g, 0, 0)),
            pl.BlockSpec((_BB, 2, K), lambda g: (g, 0, 0)),
            pl.BlockSpec((_BB, 2, K), lambda g: (g, 0, 0)),
            pl.BlockSpec((_BB, 2, K), lambda g: (g, 0, 0)),
            pl.BlockSpec((_BB, 2, K, K), lambda g: (g, 0, 0, 0)),
            pl.BlockSpec((3, _BB, 2, K, C), lambda g: (0, g, 0, 0, 0)),
            pl.BlockSpec((3, _BB, 2, K, C), lambda g: (0, g, 0, 0, 0)),
        ],
        out_specs=pl.BlockSpec((16, 128), lambda g: (0, 0)),
        out_shape=jax.ShapeDtypeStruct((16, 128), jnp.float32),
        compiler_params=pltpu.CompilerParams(
            dimension_semantics=("arbitrary",)),
    )(transform, refp, srcp, refm, srcm, refs, srcs, match, reff, srcf)

    v = out[:, 0]
    s1, c1, s2, c2, s3, c3, pos_s, neg_s, cneg = (
        v[0], v[1], v[2], v[3], v[4], v[5], v[6], v[7], v[8])
    fine_ri_loss = -(s1 / jnp.maximum(c1, 1.0)
                     + 0.5 * s2 / jnp.maximum(c2, 1.0)
                     + 0.5 * s3 / jnp.maximum(c3, 1.0))
    Cf = jnp.float32(C)
    pos_loss = pos_s / jnp.maximum(c1 * Cf, 1.0)
    neg_loss = neg_s / jnp.maximum(cneg * Cf, 1.0)
    fine_re_loss = jnp.where(c1 == 0.0, 0.0, pos_loss + neg_loss)
    return (fine_ri_loss, fine_re_loss)


# speedup vs baseline: 1.5833x; 1.0389x over previous
import jax
import jax.numpy as jnp
from jax import lax
from jax.experimental import pallas as pl
from jax.experimental.pallas import tpu as pltpu

POS_RADIUS2 = 4.0
NEG_RADIUS2 = 6.25
POS_MARGIN = 0.1
NEG_MARGIN = 1.4

_BB = 8


def _body(tr, refp, srcp, refm, srcm, refs, srcs, match, reff, srcf, out):
    g = pl.program_id(0)

    R = [[tr[l, k] for k in range(3)] for l in range(3)]
    t = [tr[l, 3] for l in range(3)]

    rp = [refp[k] for k in range(3)]
    sp_in = [srcp[k] for k in range(3)]
    sp = [sp_in[0] * R[l][0] + sp_in[1] * R[l][1] + sp_in[2] * R[l][2] + t[l]
          for l in range(3)]
    sq_r = rp[0] * rp[0] + rp[1] * rp[1] + rp[2] * rp[2]
    sq_s = sp[0] * sp[0] + sp[1] * sp[1] + sp[2] * sp[2]
    ab = (rp[0][..., :, None] * sp[0][..., None, :]
          + rp[1][..., :, None] * sp[1][..., None, :]
          + rp[2][..., :, None] * sp[2][..., None, :])
    d = sq_r[..., :, None] + sq_s[..., None, :] - 2.0 * ab

    refm_v = refm[...]
    srcm_v = srcm[...]
    gtm = refm_v[..., :, None] * srcm_v[..., None, :]
    gt = jnp.where(d < POS_RADIUS2, gtm, 0.0)
    neg = jnp.where(d > NEG_RADIUS2, gtm, 0.0)

    row = jnp.sum(gt, axis=3)
    col = jnp.sum(gt, axis=2)
    slack_r = jnp.where(row == 0.0, refm_v, 0.0)
    slack_c = jnp.where(col == 0.0, srcm_v, 0.0)

    s1 = jnp.sum(jnp.log(jnp.where(gt > 0.0, match[...], 1.0)))
    c1 = jnp.sum(gt)
    s2 = jnp.sum(jnp.log(jnp.where(slack_r > 0.0, 1.0 - refs[...], 1.0)))
    c2 = jnp.sum(slack_r)
    s3 = jnp.sum(jnp.log(jnp.where(slack_c > 0.0, 1.0 - srcs[...], 1.0)))
    c3 = jnp.sum(slack_c)
    cneg = jnp.sum(neg)

    rf = [jnp.concatenate([reff[k, :, 0], reff[k, :, 1]], axis=-1)
          for k in range(3)]
    sf_in = [jnp.concatenate([srcf[k, :, 0], srcf[k, :, 1]], axis=-1)
             for k in range(3)]
    sf = [sf_in[0] * R[l][0] + sf_in[1] * R[l][1] + sf_in[2] * R[l][2]
          for l in range(3)]
    d0 = rf[0][:, :, None, :] - sf[0][:, None, :, :]
    d1 = rf[1][:, :, None, :] - sf[1][:, None, :, :]
    d2 = rf[2][:, :, None, :] - sf[2][:, None, :, :]
    sqd = d0 * d0 + d1 * d1 + d2 * d2
    norms = sqd * lax.rsqrt(jnp.maximum(sqd, 1e-12))

    BB, K, _, C2 = sqd.shape
    C = C2 // 2
    hp = jnp.maximum(norms - POS_MARGIN, 0.0).reshape(BB, K * K, C2)
    hn = jnp.maximum(NEG_MARGIN - norms, 0.0).reshape(BB, K * K, C2)
    gtf = gt.reshape(BB, 2, K * K).astype(jnp.bfloat16)
    negf = neg.reshape(BB, 2, K * K).astype(jnp.bfloat16)
    dn = (((2,), (1,)), ((0,), (0,)))
    P = lax.dot_general(gtf, hp.astype(jnp.bfloat16), dn,
                        preferred_element_type=jnp.float32)
    N = lax.dot_general(negf, hn.astype(jnp.bfloat16), dn,
                        preferred_element_type=jnp.float32)
    lanei = lax.broadcasted_iota(jnp.int32, (BB, 2, C2), 2)
    rowi = lax.broadcasted_iota(jnp.int32, (BB, 2, C2), 1)
    sel = (lanei // C) == rowi
    pos_s = jnp.sum(jnp.where(sel, P, 0.0))
    neg_s = jnp.sum(jnp.where(sel, N, 0.0))

    @pl.when(g == 0)
    def _():
        out[...] = jnp.zeros_like(out)

    out[0:1, :] = out[0:1, :] + s1
    out[1:2, :] = out[1:2, :] + c1
    out[2:3, :] = out[2:3, :] + s2
    out[3:4, :] = out[3:4, :] + c2
    out[4:5, :] = out[4:5, :] + s3
    out[5:6, :] = out[5:6, :] + c3
    out[6:7, :] = out[6:7, :] + pos_s
    out[7:8, :] = out[7:8, :] + neg_s
    out[8:9, :] = out[8:9, :] + cneg


def kernel(ref_node_corr_knn_points, src_node_corr_knn_points,
           ref_node_corr_knn_masks, src_node_corr_knn_masks,
           ref_node_corr_knn_scores, src_node_corr_knn_scores,
           matching_scores, transform,
           re_ref_node_corr_knn_feats, re_src_node_corr_knn_feats):
    B, K, C, _ = re_ref_node_corr_knn_feats.shape
    B2 = B // 2
    G = B2 // _BB

    def fold_feats(f):
        return f.transpose(3, 0, 1, 2).reshape(3, B2, 2, K, C)

    refp = ref_node_corr_knn_points.transpose(2, 0, 1).reshape(3, B2, 2, K)
    srcp = src_node_corr_knn_points.transpose(2, 0, 1).reshape(3, B2, 2, K)
    refm = ref_node_corr_knn_masks.astype(jnp.float32).reshape(B2, 2, K)
    srcm = src_node_corr_knn_masks.astype(jnp.float32).reshape(B2, 2, K)
    refs = ref_node_corr_knn_scores.reshape(B2, 2, K)
    srcs = src_node_corr_knn_scores.reshape(B2, 2, K)
    match = matching_scores.reshape(B2, 2, K, K)
    reff = fold_feats(re_ref_node_corr_knn_feats)
    srcf = fold_feats(re_src_node_corr_knn_feats)

    out = pl.pallas_call(
        _body,
        grid=(G,),
        in_specs=[
            pl.BlockSpec(memory_space=pltpu.SMEM),
            pl.BlockSpec((3, _BB, 2, K), lambda g: (0, g, 0, 0)),
            pl.BlockSpec((3, _BB, 2, K), lambda g: (0, g, 0, 0)),
            pl.BlockSpec((_BB, 2, K), lambda g: (g, 0, 0)),
            pl.BlockSpec((_BB, 2, K), lambda g: (g, 0, 0)),
            pl.BlockSpec((_BB, 2, K), lambda g: (g, 0, 0)),
            pl.BlockSpec((_BB, 2, K), lambda g: (g, 0, 0)),
            pl.BlockSpec((_BB, 2, K, K), lambda g: (g, 0, 0, 0)),
            pl.BlockSpec((3, _BB, 2, K, C), lambda g: (0, g, 0, 0, 0)),
            pl.BlockSpec((3, _BB, 2, K, C), lambda g: (0, g, 0, 0, 0)),
        ],
        out_specs=pl.BlockSpec((16, 128), lambda g: (0, 0)),
        out_shape=jax.ShapeDtypeStruct((16, 128), jnp.float32),
        compiler_params=pltpu.CompilerParams(
            dimension_semantics=("arbitrary",)),
    )(transform, refp, srcp, refm, srcm, refs, srcs, match, reff, srcf)

    v = out[:, 0]
    s1, c1, s2, c2, s3, c3, pos_s, neg_s, cneg = (
        v[0], v[1], v[2], v[3], v[4], v[5], v[6], v[7], v[8])
    fine_ri_loss = -(s1 / jnp.maximum(c1, 1.0)
                     + 0.5 * s2 / jnp.maximum(c2, 1.0)
                     + 0.5 * s3 / jnp.maximum(c3, 1.0))
    Cf = jnp.float32(C)
    pos_loss = pos_s / jnp.maximum(c1 * Cf, 1.0)
    neg_loss = neg_s / jnp.maximum(cneg * Cf, 1.0)
    fine_re_loss = jnp.where(c1 == 0.0, 0.0, pos_loss + neg_loss)
    return (fine_ri_loss, fine_re_loss)


# speedup vs baseline: 1.6288x; 1.0287x over previous
import jax
import jax.numpy as jnp
from jax import lax
from jax.experimental import pallas as pl
from jax.experimental.pallas import tpu as pltpu

POS_RADIUS2 = 4.0
NEG_RADIUS2 = 6.25
POS_MARGIN = 0.1
NEG_MARGIN = 1.4

_BB = 16


def _body(tr, refp, srcp, refm, srcm, refs, srcs, match, reff, srcf, out):
    g = pl.program_id(0)

    R = [[tr[l, k] for k in range(3)] for l in range(3)]
    t = [tr[l, 3] for l in range(3)]

    rp = [refp[k] for k in range(3)]
    sp_in = [srcp[k] for k in range(3)]
    sp = [sp_in[0] * R[l][0] + sp_in[1] * R[l][1] + sp_in[2] * R[l][2] + t[l]
          for l in range(3)]
    sq_r = rp[0] * rp[0] + rp[1] * rp[1] + rp[2] * rp[2]
    sq_s = sp[0] * sp[0] + sp[1] * sp[1] + sp[2] * sp[2]
    ab = (rp[0][..., :, None] * sp[0][..., None, :]
          + rp[1][..., :, None] * sp[1][..., None, :]
          + rp[2][..., :, None] * sp[2][..., None, :])
    d = sq_r[..., :, None] + sq_s[..., None, :] - 2.0 * ab

    refm_v = refm[...]
    srcm_v = srcm[...]
    gtm = refm_v[..., :, None] * srcm_v[..., None, :]
    gt = jnp.where(d < POS_RADIUS2, gtm, 0.0)
    neg = jnp.where(d > NEG_RADIUS2, gtm, 0.0)

    row = jnp.sum(gt, axis=3)
    col = jnp.sum(gt, axis=2)
    slack_r = jnp.where(row == 0.0, refm_v, 0.0)
    slack_c = jnp.where(col == 0.0, srcm_v, 0.0)

    s1 = jnp.sum(jnp.log(jnp.where(gt > 0.0, match[...], 1.0)))
    c1 = jnp.sum(gt)
    s2 = jnp.sum(jnp.log(jnp.where(slack_r > 0.0, 1.0 - refs[...], 1.0)))
    c2 = jnp.sum(slack_r)
    s3 = jnp.sum(jnp.log(jnp.where(slack_c > 0.0, 1.0 - srcs[...], 1.0)))
    c3 = jnp.sum(slack_c)
    cneg = jnp.sum(neg)

    rf = [jnp.concatenate([reff[k, :, 0], reff[k, :, 1]], axis=-1)
          for k in range(3)]
    sf_in = [jnp.concatenate([srcf[k, :, 0], srcf[k, :, 1]], axis=-1)
             for k in range(3)]
    sf = [sf_in[0] * R[l][0] + sf_in[1] * R[l][1] + sf_in[2] * R[l][2]
          for l in range(3)]
    d0 = rf[0][:, :, None, :] - sf[0][:, None, :, :]
    d1 = rf[1][:, :, None, :] - sf[1][:, None, :, :]
    d2 = rf[2][:, :, None, :] - sf[2][:, None, :, :]
    sqd = d0 * d0 + d1 * d1 + d2 * d2
    norms = sqd * lax.rsqrt(jnp.maximum(sqd, 1e-12))

    BB, K, _, C2 = sqd.shape
    C = C2 // 2
    hp = jnp.maximum(norms - POS_MARGIN, 0.0).reshape(BB, K * K, C2)
    hn = jnp.maximum(NEG_MARGIN - norms, 0.0).reshape(BB, K * K, C2)
    gtf = gt.reshape(BB, 2, K * K).astype(jnp.bfloat16)
    negf = neg.reshape(BB, 2, K * K).astype(jnp.bfloat16)
    dn = (((2,), (1,)), ((0,), (0,)))
    P = lax.dot_general(gtf, hp.astype(jnp.bfloat16), dn,
                        preferred_element_type=jnp.float32)
    N = lax.dot_general(negf, hn.astype(jnp.bfloat16), dn,
                        preferred_element_type=jnp.float32)
    lanei = lax.broadcasted_iota(jnp.int32, (BB, 2, C2), 2)
    rowi = lax.broadcasted_iota(jnp.int32, (BB, 2, C2), 1)
    sel = (lanei // C) == rowi
    pos_s = jnp.sum(jnp.where(sel, P, 0.0))
    neg_s = jnp.sum(jnp.where(sel, N, 0.0))

    @pl.when(g == 0)
    def _():
        out[...] = jnp.zeros_like(out)

    out[0:1, :] = out[0:1, :] + s1
    out[1:2, :] = out[1:2, :] + c1
    out[2:3, :] = out[2:3, :] + s2
    out[3:4, :] = out[3:4, :] + c2
    out[4:5, :] = out[4:5, :] + s3
    out[5:6, :] = out[5:6, :] + c3
    out[6:7, :] = out[6:7, :] + pos_s
    out[7:8, :] = out[7:8, :] + neg_s
    out[8:9, :] = out[8:9, :] + cneg


def kernel(ref_node_corr_knn_points, src_node_corr_knn_points,
           ref_node_corr_knn_masks, src_node_corr_knn_masks,
           ref_node_corr_knn_scores, src_node_corr_knn_scores,
           matching_scores, transform,
           re_ref_node_corr_knn_feats, re_src_node_corr_knn_feats):
    B, K, C, _ = re_ref_node_corr_knn_feats.shape
    B2 = B // 2
    G = B2 // _BB

    def fold_feats(f):
        return f.transpose(3, 0, 1, 2).reshape(3, B2, 2, K, C)

    refp = ref_node_corr_knn_points.transpose(2, 0, 1).reshape(3, B2, 2, K)
    srcp = src_node_corr_knn_points.transpose(2, 0, 1).reshape(3, B2, 2, K)
    refm = ref_node_corr_knn_masks.astype(jnp.float32).reshape(B2, 2, K)
    srcm = src_node_corr_knn_masks.astype(jnp.float32).reshape(B2, 2, K)
    refs = ref_node_corr_knn_scores.reshape(B2, 2, K)
    srcs = src_node_corr_knn_scores.reshape(B2, 2, K)
    match = matching_scores.reshape(B2, 2, K, K)
    reff = fold_feats(re_ref_node_corr_knn_feats)
    srcf = fold_feats(re_src_node_corr_knn_feats)

    out = pl.pallas_call(
        _body,
        grid=(G,),
        in_specs=[
            pl.BlockSpec(memory_space=pltpu.SMEM),
            pl.BlockSpec((3, _BB, 2, K), lambda g: (0, g, 0, 0)),
            pl.BlockSpec((3, _BB, 2, K), lambda g: (0, g, 0, 0)),
            pl.BlockSpec((_BB, 2, K), lambda g: (g, 0, 0)),
            pl.BlockSpec((_BB, 2, K), lambda g: (g, 0, 0)),
            pl.BlockSpec((_BB, 2, K), lambda g: (g, 0, 0)),
            pl.BlockSpec((_BB, 2, K), lambda g: (g, 0, 0)),
            pl.BlockSpec((_BB, 2, K, K), lambda g: (g, 0, 0, 0)),
            pl.BlockSpec((3, _BB, 2, K, C), lambda g: (0, g, 0, 0, 0)),
            pl.BlockSpec((3, _BB, 2, K, C), lambda g: (0, g, 0, 0, 0)),
        ],
        out_specs=pl.BlockSpec((16, 128), lambda g: (0, 0)),
        out_shape=jax.ShapeDtypeStruct((16, 128), jnp.float32),
        compiler_params=pltpu.CompilerParams(
            dimension_semantics=("arbitrary",)),
    )(transform, refp, srcp, refm, srcm, refs, srcs, match, reff, srcf)

    v = out[:, 0]
    s1, c1, s2, c2, s3, c3, pos_s, neg_s, cneg = (
        v[0], v[1], v[2], v[3], v[4], v[5], v[6], v[7], v[8])
    fine_ri_loss = -(s1 / jnp.maximum(c1, 1.0)
                     + 0.5 * s2 / jnp.maximum(c2, 1.0)
                     + 0.5 * s3 / jnp.maximum(c3, 1.0))
    Cf = jnp.float32(C)
    pos_loss = pos_s / jnp.maximum(c1 * Cf, 1.0)
    neg_loss = neg_s / jnp.maximum(cneg * Cf, 1.0)
    fine_re_loss = jnp.where(c1 == 0.0, 0.0, pos_loss + neg_loss)
    return (fine_ri_loss, fine_re_loss)
